# baseline (speedup 1.0000x reference)
import functools

import jax
import jax.numpy as jnp
from jax import lax
from jax.experimental import pallas as pl
from jax.experimental.pallas import tpu as pltpu, tpu_sc as plsc

N = 50000
NS = 2048
NTILES = 16
NPAD = 50176
PPT = NPAD // NTILES
UNROLL = 14
GROUPS = PPT // (16 * UNROLL)
IMAX = 2147483647

_mesh = plsc.VectorSubcoreMesh(
    core_axis_name="c", subcore_axis_name="s", num_cores=1
)


@functools.partial(
    pl.kernel,
    mesh=_mesh,
    out_type=jax.ShapeDtypeStruct((NS, 1, 16), jnp.float32),
    compiler_params=pltpu.CompilerParams(
        needs_layout_passes=False, use_tc_tiling_on_sc=False),
    scratch_types=[
        pltpu.VMEM((3, PPT + 16), jnp.float32),
        pltpu.VMEM((1, PPT), jnp.float32),
        pltpu.VMEM((1, 1, 16), jnp.float32),
        pltpu.VMEM((16, 1, 48), jnp.float32),
        pltpu.VMEM((1, 48), jnp.float32),
        pltpu.VMEM((NS, 1, 16), jnp.float32),
        pltpu.VMEM_SHARED((32, 1, 48), jnp.float32),
    ],
)
def _fps_sc(xt_hbm, dinit_hbm, x0_hbm, out_hbm,
            xloc, dists, p0buf, redbuf, recstage, outbuf, recs):
    sid = lax.axis_index("s")

    pltpu.sync_copy(xt_hbm.at[sid], xloc)
    pltpu.sync_copy(dinit_hbm.at[sid], dists)
    pltpu.sync_copy(x0_hbm, p0buf)

    pv = p0buf[0, 0, :]

    @pl.when(sid == 0)
    def _():
        outbuf[0, 0, :] = pv

    base = sid * PPT
    lanes = lax.iota(jnp.int32, 16)
    imax16 = jnp.full((16,), IMAX, jnp.int32)
    ninf16 = jnp.full((16,), -jnp.inf, jnp.float32)
    zeros16 = jnp.zeros((16,), jnp.int32)

    def iter_body(i, carry):
        p0, p1, p2 = carry

        def chunk(off, acc):
            runmax, runidx = acc
            t0 = xloc[0, pl.ds(off, 16)] - p0
            t1 = xloc[1, pl.ds(off, 16)] - p1
            t2 = xloc[2, pl.ds(off, 16)] - p2
            d = (t0 * t0 + t2 * t2) + t1 * t1
            dn = jnp.minimum(dists[0, pl.ds(off, 16)], d)
            dists[0, pl.ds(off, 16)] = dn
            upd = dn > runmax
            runmax = jnp.maximum(runmax, dn)
            runidx = jnp.where(upd, (base + off) + lanes, runidx)
            return runmax, runidx

        def group(g, accs):
            off = g * (16 * UNROLL)
            return tuple(
                chunk(off + 16 * u, accs[u]) for u in range(UNROLL))

        acc0 = (ninf16, zeros16)
        accs = lax.fori_loop(
            0, GROUPS, group, tuple(acc0 for _ in range(UNROLL)))

        def merge(a, b):
            va, ia = a
            vb, ib = b
            take = (vb > va) | ((vb == va) & (ib < ia))
            return jnp.where(take, vb, va), jnp.where(take, ib, ia)

        acc = accs[0]
        for u in range(1, UNROLL):
            acc = merge(acc, accs[u])
        runmax, runidx = acc

        m = jnp.max(runmax)
        mvec = jnp.full((16,), m, jnp.float32)
        li = jnp.min(jnp.where(runmax == mvec, runidx, imax16))
        lo = li - base
        cx = xloc[0, pl.ds(lo, 16)]
        cy = xloc[1, pl.ds(lo, 16)]
        cz = xloc[2, pl.ds(lo, 16)]
        coords = jnp.where(lanes == 0, jnp.full((16,), cx[0], jnp.float32),
                 jnp.where(lanes == 1, jnp.full((16,), cy[0], jnp.float32),
                           jnp.full((16,), cz[0], jnp.float32)))
        recstage[0, pl.ds(0, 16)] = mvec
        recstage[0, pl.ds(16, 16)] = plsc.bitcast(
            jnp.full((16,), li, jnp.int32), jnp.float32)
        recstage[0, pl.ds(32, 16)] = coords

        pbase = jnp.bitwise_and(i, 1) * NTILES
        pltpu.sync_copy(recstage, recs.at[pbase + sid])
        plsc.subcore_barrier()
        pltpu.sync_copy(recs.at[pl.ds(pbase, NTILES)], redbuf)

        bestv = redbuf[0, 0, pl.ds(0, 16)]
        besti = plsc.bitcast(redbuf[0, 0, pl.ds(16, 16)], jnp.int32)
        bestc = redbuf[0, 0, pl.ds(32, 16)]
        for t in range(1, NTILES):
            vt = redbuf[t, 0, pl.ds(0, 16)]
            it = plsc.bitcast(redbuf[t, 0, pl.ds(16, 16)], jnp.int32)
            ct = redbuf[t, 0, pl.ds(32, 16)]
            take = (vt > bestv) | ((vt == bestv) & (it < besti))
            bestv = jnp.where(take, vt, bestv)
            besti = jnp.where(take, it, besti)
            bestc = jnp.where(take, ct, bestc)

        @pl.when(sid == 0)
        def _():
            outbuf[i, 0, :] = bestc

        return (jnp.full((16,), bestc[0], jnp.float32),
                jnp.full((16,), bestc[1], jnp.float32),
                jnp.full((16,), bestc[2], jnp.float32))

    lax.fori_loop(
        1, NS, iter_body,
        (jnp.full((16,), pv[0], jnp.float32),
         jnp.full((16,), pv[1], jnp.float32),
         jnp.full((16,), pv[2], jnp.float32)),
    )

    @pl.when(sid == 0)
    def _():
        pltpu.sync_copy(outbuf, out_hbm)


def kernel(x):
    xpad = jnp.pad(x, ((0, NPAD - N), (0, 0)))
    xt = xpad.T.reshape(3, NTILES, PPT).transpose(1, 0, 2)
    xt = jnp.pad(xt, ((0, 0), (0, 0), (0, 16)))
    x0 = jnp.pad(x[0:1], ((0, 0), (0, 13))).reshape(1, 1, 16)
    dinit = jnp.concatenate([
        jnp.full((N,), jnp.inf, jnp.float32),
        jnp.full((NPAD - N,), -jnp.inf, jnp.float32),
    ]).reshape(NTILES, 1, PPT)
    out = _fps_sc(xt, dinit, x0)
    return out.reshape(NS, 16)[:, :3]


# speedup vs baseline: 2.6469x; 2.6469x over previous
import functools

import jax
import jax.numpy as jnp
from jax import lax
from jax.experimental import pallas as pl
from jax.experimental.pallas import tpu as pltpu, tpu_sc as plsc

N = 50000
NS = 2048
NTILES = 16
NPAD = 50176
PPT = NPAD // NTILES
UNROLL = 2
GROUPS = PPT // (16 * UNROLL)
IMAX = 2147483647

_mesh = plsc.VectorSubcoreMesh(
    core_axis_name="c", subcore_axis_name="s", num_cores=1
)


@functools.partial(
    pl.kernel,
    mesh=_mesh,
    out_type=jax.ShapeDtypeStruct((NS, 1, 16), jnp.float32),
    compiler_params=pltpu.CompilerParams(
        needs_layout_passes=False, use_tc_tiling_on_sc=False),
    scratch_types=[
        pltpu.VMEM((3, PPT + 16), jnp.float32),
        pltpu.VMEM((1, PPT), jnp.float32),
        pltpu.VMEM((1, 1, 16), jnp.float32),
        pltpu.VMEM((16, 1, 48), jnp.float32),
        pltpu.VMEM((1, 48), jnp.float32),
        pltpu.VMEM((NS, 1, 16), jnp.float32),
        pltpu.VMEM_SHARED((32, 1, 48), jnp.float32),
    ],
)
def _fps_sc(xt_hbm, dinit_hbm, x0_hbm, out_hbm,
            xloc, dists, p0buf, redbuf, recstage, outbuf, recs):
    sid = lax.axis_index("s")

    pltpu.sync_copy(xt_hbm.at[sid], xloc)
    pltpu.sync_copy(dinit_hbm.at[sid], dists)
    pltpu.sync_copy(x0_hbm, p0buf)

    pv = p0buf[0, 0, :]

    @pl.when(sid == 0)
    def _():
        outbuf[0, 0, :] = pv

    base = sid * PPT
    lanes = lax.iota(jnp.int32, 16)
    imax16 = jnp.full((16,), IMAX, jnp.int32)
    ninf16 = jnp.full((16,), -jnp.inf, jnp.float32)
    zeros16 = jnp.zeros((16,), jnp.int32)

    def iter_body(i, carry):
        p0, p1, p2 = carry

        def chunk(off, acc):
            runmax, runidx = acc
            t0 = xloc[0, pl.ds(off, 16)] - p0
            t1 = xloc[1, pl.ds(off, 16)] - p1
            t2 = xloc[2, pl.ds(off, 16)] - p2
            d = (t0 * t0 + t2 * t2) + t1 * t1
            dn = jnp.minimum(dists[0, pl.ds(off, 16)], d)
            dists[0, pl.ds(off, 16)] = dn
            upd = dn > runmax
            runmax = jnp.maximum(runmax, dn)
            runidx = jnp.where(upd, (base + off) + lanes, runidx)
            return runmax, runidx

        def group(g, accs):
            off = g * (16 * UNROLL)
            return tuple(
                chunk(off + 16 * u, accs[u]) for u in range(UNROLL))

        acc0 = (ninf16, zeros16)
        accs = lax.fori_loop(
            0, GROUPS, group, tuple(acc0 for _ in range(UNROLL)))

        def merge(a, b):
            va, ia = a
            vb, ib = b
            take = (vb > va) | ((vb == va) & (ib < ia))
            return jnp.where(take, vb, va), jnp.where(take, ib, ia)

        acc = accs[0]
        for u in range(1, UNROLL):
            acc = merge(acc, accs[u])
        runmax, runidx = acc

        m = jnp.max(runmax)
        mvec = jnp.full((16,), m, jnp.float32)
        li = jnp.min(jnp.where(runmax == mvec, runidx, imax16))
        lo = li - base
        cx = xloc[0, pl.ds(lo, 16)]
        cy = xloc[1, pl.ds(lo, 16)]
        cz = xloc[2, pl.ds(lo, 16)]
        coords = jnp.where(lanes == 0, jnp.full((16,), cx[0], jnp.float32),
                 jnp.where(lanes == 1, jnp.full((16,), cy[0], jnp.float32),
                           jnp.full((16,), cz[0], jnp.float32)))
        recstage[0, pl.ds(0, 16)] = mvec
        recstage[0, pl.ds(16, 16)] = plsc.bitcast(
            jnp.full((16,), li, jnp.int32), jnp.float32)
        recstage[0, pl.ds(32, 16)] = coords

        pbase = jnp.bitwise_and(i, 1) * NTILES
        pltpu.sync_copy(recstage, recs.at[pbase + sid])
        plsc.subcore_barrier()
        pltpu.sync_copy(recs.at[pl.ds(pbase, NTILES)], redbuf)

        bestv = redbuf[0, 0, pl.ds(0, 16)]
        besti = plsc.bitcast(redbuf[0, 0, pl.ds(16, 16)], jnp.int32)
        bestc = redbuf[0, 0, pl.ds(32, 16)]
        for t in range(1, NTILES):
            vt = redbuf[t, 0, pl.ds(0, 16)]
            it = plsc.bitcast(redbuf[t, 0, pl.ds(16, 16)], jnp.int32)
            ct = redbuf[t, 0, pl.ds(32, 16)]
            take = (vt > bestv) | ((vt == bestv) & (it < besti))
            bestv = jnp.where(take, vt, bestv)
            besti = jnp.where(take, it, besti)
            bestc = jnp.where(take, ct, bestc)

        @pl.when(sid == 0)
        def _():
            outbuf[i, 0, :] = bestc

        return (jnp.full((16,), bestc[0], jnp.float32),
                jnp.full((16,), bestc[1], jnp.float32),
                jnp.full((16,), bestc[2], jnp.float32))

    lax.fori_loop(
        1, NS, iter_body,
        (jnp.full((16,), pv[0], jnp.float32),
         jnp.full((16,), pv[1], jnp.float32),
         jnp.full((16,), pv[2], jnp.float32)),
    )

    @pl.when(sid == 0)
    def _():
        pltpu.sync_copy(outbuf, out_hbm)


def kernel(x):
    xpad = jnp.pad(x, ((0, NPAD - N), (0, 0)))
    xt = xpad.T.reshape(3, NTILES, PPT).transpose(1, 0, 2)
    xt = jnp.pad(xt, ((0, 0), (0, 0), (0, 16)))
    x0 = jnp.pad(x[0:1], ((0, 0), (0, 13))).reshape(1, 1, 16)
    dinit = jnp.concatenate([
        jnp.full((N,), jnp.inf, jnp.float32),
        jnp.full((NPAD - N,), -jnp.inf, jnp.float32),
    ]).reshape(NTILES, 1, PPT)
    out = _fps_sc(xt, dinit, x0)
    return out.reshape(NS, 16)[:, :3]


# speedup vs baseline: 2.6548x; 1.0030x over previous
import functools

import jax
import jax.numpy as jnp
from jax import lax
from jax.experimental import pallas as pl
from jax.experimental.pallas import tpu as pltpu, tpu_sc as plsc

N = 50000
NS = 2048
NTILES = 16
NPAD = 50176
PPT = NPAD // NTILES
UNROLL = 2
GROUPS = PPT // (16 * UNROLL)
IMAX = 2147483647

_mesh = plsc.VectorSubcoreMesh(
    core_axis_name="c", subcore_axis_name="s", num_cores=1
)


@functools.partial(
    pl.kernel,
    mesh=_mesh,
    out_type=jax.ShapeDtypeStruct((NS, 1, 16), jnp.float32),
    compiler_params=pltpu.CompilerParams(
        needs_layout_passes=False, use_tc_tiling_on_sc=False),
    scratch_types=[
        pltpu.VMEM((3, PPT + 16), jnp.float32),
        pltpu.VMEM((1, PPT), jnp.float32),
        pltpu.VMEM((1, 1, 16), jnp.float32),
        pltpu.VMEM((16, 1, 48), jnp.float32),
        pltpu.VMEM((1, 48), jnp.float32),
        pltpu.VMEM((NS, 1, 16), jnp.float32),
        pltpu.VMEM_SHARED((32, 1, 48), jnp.float32),
    ],
)
def _fps_sc(xt_hbm, dinit_hbm, x0_hbm, out_hbm,
            xloc, dists, p0buf, redbuf, recstage, outbuf, recs):
    sid = lax.axis_index("s")

    pltpu.sync_copy(xt_hbm.at[sid], xloc)
    pltpu.sync_copy(dinit_hbm.at[sid], dists)
    pltpu.sync_copy(x0_hbm, p0buf)

    pv = p0buf[0, 0, :]

    @pl.when(sid == 0)
    def _():
        outbuf[0, 0, :] = pv

    base = sid * PPT
    lanes = lax.iota(jnp.int32, 16)
    imax16 = jnp.full((16,), IMAX, jnp.int32)
    ninf16 = jnp.full((16,), -jnp.inf, jnp.float32)
    zeros16 = jnp.zeros((16,), jnp.int32)

    def iter_body(i, carry):
        p0, p1, p2 = carry

        def chunk(off, acc):
            runmax, runidx = acc
            t0 = xloc[0, pl.ds(off, 16)] - p0
            t1 = xloc[1, pl.ds(off, 16)] - p1
            t2 = xloc[2, pl.ds(off, 16)] - p2
            d = (t0 * t0 + t2 * t2) + t1 * t1
            dn = jnp.minimum(dists[0, pl.ds(off, 16)], d)
            dists[0, pl.ds(off, 16)] = dn
            upd = dn > runmax
            runmax = jnp.maximum(runmax, dn)
            runidx = jnp.where(upd, (base + off) + lanes, runidx)
            return runmax, runidx

        acc0 = (ninf16, zeros16)

        @plsc.parallel_loop(0, PPT // 16, 1, unroll=UNROLL, carry=acc0)
        def final_acc(c, acc):
            return chunk(c * 16, acc)

        runmax, runidx = final_acc

        m = jnp.max(runmax)
        mvec = jnp.full((16,), m, jnp.float32)
        li = jnp.min(jnp.where(runmax == mvec, runidx, imax16))
        lo = li - base
        cx = xloc[0, pl.ds(lo, 16)]
        cy = xloc[1, pl.ds(lo, 16)]
        cz = xloc[2, pl.ds(lo, 16)]
        coords = jnp.where(lanes == 0, jnp.full((16,), cx[0], jnp.float32),
                 jnp.where(lanes == 1, jnp.full((16,), cy[0], jnp.float32),
                           jnp.full((16,), cz[0], jnp.float32)))
        recstage[0, pl.ds(0, 16)] = mvec
        recstage[0, pl.ds(16, 16)] = plsc.bitcast(
            jnp.full((16,), li, jnp.int32), jnp.float32)
        recstage[0, pl.ds(32, 16)] = coords

        pbase = jnp.bitwise_and(i, 1) * NTILES
        pltpu.sync_copy(recstage, recs.at[pbase + sid])
        plsc.subcore_barrier()
        pltpu.sync_copy(recs.at[pl.ds(pbase, NTILES)], redbuf)

        bestv = redbuf[0, 0, pl.ds(0, 16)]
        besti = plsc.bitcast(redbuf[0, 0, pl.ds(16, 16)], jnp.int32)
        bestc = redbuf[0, 0, pl.ds(32, 16)]
        for t in range(1, NTILES):
            vt = redbuf[t, 0, pl.ds(0, 16)]
            it = plsc.bitcast(redbuf[t, 0, pl.ds(16, 16)], jnp.int32)
            ct = redbuf[t, 0, pl.ds(32, 16)]
            take = (vt > bestv) | ((vt == bestv) & (it < besti))
            bestv = jnp.where(take, vt, bestv)
            besti = jnp.where(take, it, besti)
            bestc = jnp.where(take, ct, bestc)

        @pl.when(sid == 0)
        def _():
            outbuf[i, 0, :] = bestc

        return (jnp.full((16,), bestc[0], jnp.float32),
                jnp.full((16,), bestc[1], jnp.float32),
                jnp.full((16,), bestc[2], jnp.float32))

    lax.fori_loop(
        1, NS, iter_body,
        (jnp.full((16,), pv[0], jnp.float32),
         jnp.full((16,), pv[1], jnp.float32),
         jnp.full((16,), pv[2], jnp.float32)),
    )

    @pl.when(sid == 0)
    def _():
        pltpu.sync_copy(outbuf, out_hbm)


def kernel(x):
    xpad = jnp.pad(x, ((0, NPAD - N), (0, 0)))
    xt = xpad.T.reshape(3, NTILES, PPT).transpose(1, 0, 2)
    xt = jnp.pad(xt, ((0, 0), (0, 0), (0, 16)))
    x0 = jnp.pad(x[0:1], ((0, 0), (0, 13))).reshape(1, 1, 16)
    dinit = jnp.concatenate([
        jnp.full((N,), jnp.inf, jnp.float32),
        jnp.full((NPAD - N,), -jnp.inf, jnp.float32),
    ]).reshape(NTILES, 1, PPT)
    out = _fps_sc(xt, dinit, x0)
    return out.reshape(NS, 16)[:, :3]


# speedup vs baseline: 2.6691x; 1.0054x over previous
import functools

import jax
import jax.numpy as jnp
from jax import lax
from jax.experimental import pallas as pl
from jax.experimental.pallas import tpu as pltpu, tpu_sc as plsc

N = 50000
NS = 2048
NTILES = 16
NPAD = 50176
PPT = NPAD // NTILES
UNROLL = 4
GROUPS = PPT // (16 * UNROLL)
IMAX = 2147483647

_mesh = plsc.VectorSubcoreMesh(
    core_axis_name="c", subcore_axis_name="s", num_cores=1
)


@functools.partial(
    pl.kernel,
    mesh=_mesh,
    out_type=jax.ShapeDtypeStruct((NS, 1, 16), jnp.float32),
    compiler_params=pltpu.CompilerParams(
        needs_layout_passes=False, use_tc_tiling_on_sc=False),
    scratch_types=[
        pltpu.VMEM((3, PPT + 16), jnp.float32),
        pltpu.VMEM((1, PPT), jnp.float32),
        pltpu.VMEM((1, 1, 16), jnp.float32),
        pltpu.VMEM((16, 1, 48), jnp.float32),
        pltpu.VMEM((1, 48), jnp.float32),
        pltpu.VMEM((NS, 1, 16), jnp.float32),
        pltpu.VMEM_SHARED((32, 1, 48), jnp.float32),
    ],
)
def _fps_sc(xt_hbm, dinit_hbm, x0_hbm, out_hbm,
            xloc, dists, p0buf, redbuf, recstage, outbuf, recs):
    sid = lax.axis_index("s")

    pltpu.sync_copy(xt_hbm.at[sid], xloc)
    pltpu.sync_copy(dinit_hbm.at[sid], dists)
    pltpu.sync_copy(x0_hbm, p0buf)

    pv = p0buf[0, 0, :]

    @pl.when(sid == 0)
    def _():
        outbuf[0, 0, :] = pv

    base = sid * PPT
    lanes = lax.iota(jnp.int32, 16)
    imax16 = jnp.full((16,), IMAX, jnp.int32)
    ninf16 = jnp.full((16,), -jnp.inf, jnp.float32)
    zeros16 = jnp.zeros((16,), jnp.int32)

    def iter_body(i, carry):
        p0, p1, p2 = carry

        def chunk(off, acc):
            runmax, runidx = acc
            t0 = xloc[0, pl.ds(off, 16)] - p0
            t1 = xloc[1, pl.ds(off, 16)] - p1
            t2 = xloc[2, pl.ds(off, 16)] - p2
            d = (t0 * t0 + t2 * t2) + t1 * t1
            dn = jnp.minimum(dists[0, pl.ds(off, 16)], d)
            dists[0, pl.ds(off, 16)] = dn
            upd = dn > runmax
            runmax = jnp.maximum(runmax, dn)
            runidx = jnp.where(upd, (base + off) + lanes, runidx)
            return runmax, runidx

        acc0 = (ninf16, zeros16)

        @plsc.parallel_loop(0, PPT // 16, 1, unroll=UNROLL, carry=acc0)
        def final_acc(c, acc):
            return chunk(c * 16, acc)

        runmax, runidx = final_acc

        m = jnp.max(runmax)
        mvec = jnp.full((16,), m, jnp.float32)
        li = jnp.min(jnp.where(runmax == mvec, runidx, imax16))
        lo = li - base
        cx = xloc[0, pl.ds(lo, 16)]
        cy = xloc[1, pl.ds(lo, 16)]
        cz = xloc[2, pl.ds(lo, 16)]
        coords = jnp.where(lanes == 0, jnp.full((16,), cx[0], jnp.float32),
                 jnp.where(lanes == 1, jnp.full((16,), cy[0], jnp.float32),
                           jnp.full((16,), cz[0], jnp.float32)))
        recstage[0, pl.ds(0, 16)] = mvec
        recstage[0, pl.ds(16, 16)] = plsc.bitcast(
            jnp.full((16,), li, jnp.int32), jnp.float32)
        recstage[0, pl.ds(32, 16)] = coords

        pbase = jnp.bitwise_and(i, 1) * NTILES
        pltpu.sync_copy(recstage, recs.at[pbase + sid])
        plsc.subcore_barrier()
        pltpu.sync_copy(recs.at[pl.ds(pbase, NTILES)], redbuf)

        bestv = redbuf[0, 0, pl.ds(0, 16)]
        besti = plsc.bitcast(redbuf[0, 0, pl.ds(16, 16)], jnp.int32)
        bestc = redbuf[0, 0, pl.ds(32, 16)]
        for t in range(1, NTILES):
            vt = redbuf[t, 0, pl.ds(0, 16)]
            it = plsc.bitcast(redbuf[t, 0, pl.ds(16, 16)], jnp.int32)
            ct = redbuf[t, 0, pl.ds(32, 16)]
            take = (vt > bestv) | ((vt == bestv) & (it < besti))
            bestv = jnp.where(take, vt, bestv)
            besti = jnp.where(take, it, besti)
            bestc = jnp.where(take, ct, bestc)

        @pl.when(sid == 0)
        def _():
            outbuf[i, 0, :] = bestc

        return (jnp.full((16,), bestc[0], jnp.float32),
                jnp.full((16,), bestc[1], jnp.float32),
                jnp.full((16,), bestc[2], jnp.float32))

    lax.fori_loop(
        1, NS, iter_body,
        (jnp.full((16,), pv[0], jnp.float32),
         jnp.full((16,), pv[1], jnp.float32),
         jnp.full((16,), pv[2], jnp.float32)),
    )

    @pl.when(sid == 0)
    def _():
        pltpu.sync_copy(outbuf, out_hbm)


def kernel(x):
    xpad = jnp.pad(x, ((0, NPAD - N), (0, 0)))
    xt = xpad.T.reshape(3, NTILES, PPT).transpose(1, 0, 2)
    xt = jnp.pad(xt, ((0, 0), (0, 0), (0, 16)))
    x0 = jnp.pad(x[0:1], ((0, 0), (0, 13))).reshape(1, 1, 16)
    dinit = jnp.concatenate([
        jnp.full((N,), jnp.inf, jnp.float32),
        jnp.full((NPAD - N,), -jnp.inf, jnp.float32),
    ]).reshape(NTILES, 1, PPT)
    out = _fps_sc(xt, dinit, x0)
    return out.reshape(NS, 16)[:, :3]
